# Initial kernel scaffold; baseline (speedup 1.0000x reference)
import jax
import jax.numpy as jnp
from jax.experimental import pallas as pl


def kernel(score, target):
    raise NotImplementedError("write your pallas kernel here")


# speedup vs baseline: 39.6215x; 39.6215x over previous
import functools

import numpy as np
import jax
import jax.numpy as jnp
from jax import lax
from jax.experimental import pallas as pl
from jax.experimental.pallas import tpu as pltpu
from jax.experimental.pallas import tpu_sc as plsc

B = 128
N = 100000
L = 16
CHUNK = 20000
NCHUNK = N // CHUNK
NVREG = CHUNK // L

_info = plsc.get_sparse_core_info()
NC, NS = _info.num_cores, _info.num_subcores
NW = NC * NS
RPW = B // NW
TOT = RPW * NCHUNK

_HARM = np.concatenate([
    np.zeros(1, np.float32),
    np.cumsum((1.0 / np.arange(1, N, dtype=np.float32)), dtype=np.float32),
]).astype(np.float32)


def _gather(vec, idx):
    return vec.at[idx].get(mode="promise_in_bounds")


def _make_kernel():
    mesh = plsc.VectorSubcoreMesh(core_axis_name="c", subcore_axis_name="s")

    @functools.partial(
        pl.kernel,
        mesh=mesh,
        compiler_params=pltpu.CompilerParams(needs_layout_passes=False),
        out_type=jax.ShapeDtypeStruct((NW, L), jnp.float32),
        scratch_types=[
            pltpu.VMEM((CHUNK,), jnp.float32),
            pltpu.VMEM((CHUNK,), jnp.float32),
            pltpu.VMEM((B,), jnp.int32),
            pltpu.VMEM((RPW, L), jnp.float32),
            pltpu.VMEM((L,), jnp.float32),
            pltpu.VMEM((L,), jnp.float32),
            pltpu.SemaphoreType.DMA,
            pltpu.SemaphoreType.DMA,
        ],
    )
    def wrpl(score_hbm, target_hbm, harm_hbm, out_hbm,
             buf0, buf1, tgt_v, gtw, hw, ov, sem0, sem1):
        wid = lax.axis_index("s") * NC + lax.axis_index("c")
        row0 = wid * RPW
        iota = lax.iota(jnp.int32, L)

        pltpu.sync_copy(target_hbm, tgt_v)

        bufs = (buf0, buf1)
        sems = (sem0, sem1)

        def issue(k, slot):
            b = row0 + (k // NCHUNK)
            off = pl.multiple_of(b * N + (k % NCHUNK) * CHUNK, 8)
            return pltpu.async_copy(score_hbm.at[pl.ds(off, CHUNK)],
                                    bufs[slot], sems[slot])

        handles = [issue(0, 0), None]

        tvs, gtvs = [], []
        for r in range(RPW):
            b = row0 + r
            wb = (b // L) * L
            twin = tgt_v[pl.ds(wb, L)]
            tv = _gather(twin, jnp.zeros((L,), jnp.int32) + (b - wb))
            t_s = jnp.max(tv)
            woff = jnp.minimum((t_s // 8) * 8, N - L)
            pltpu.sync_copy(
                score_hbm.at[pl.ds(pl.multiple_of(b * N + woff, 8), L)],
                gtw.at[r])
            gtvs.append(_gather(gtw[r], tv - woff))
            tvs.append(tv)

        contrib = jnp.zeros((L,), jnp.float32)
        for r in range(RPW):
            tv, gtv = tvs[r], gtvs[r]
            gtm1 = gtv - 1.0
            cg = jnp.zeros((L,), jnp.int32)
            ct = jnp.zeros((L,), jnp.int32)
            hh = jnp.zeros((L,), jnp.float32)
            for c in range(NCHUNK):
                k = r * NCHUNK + c
                slot = k % 2
                handles[slot].wait()
                if k + 1 < TOT:
                    handles[1 - slot] = issue(k + 1, 1 - slot)
                buf = bufs[slot]
                base = c * CHUNK

                def body(i, carry, buf=buf, base=base, tv=tv, gtv=gtv, gtm1=gtm1):
                    cg, ct, hh = carry
                    v = buf[pl.ds(i * L, L)]
                    jv = (base + i * L) + iota
                    cg = cg + (v > gtv).astype(jnp.int32)
                    ct = ct + ((v == gtv) & (jv < tv)).astype(jnp.int32)
                    hh = hh + jnp.maximum(v - gtm1, 0.0)
                    return cg, ct, hh

                cg, ct, hh = lax.fori_loop(0, NVREG, body, (cg, ct, hh))

            rank = jnp.sum(cg) + jnp.sum(ct)
            hinge = jnp.sum(hh) - 1.0
            hoff = jnp.minimum((rank // 8) * 8, N - L)
            pltpu.sync_copy(harm_hbm.at[pl.ds(pl.multiple_of(hoff, 8), L)], hw)
            rank_v = jnp.zeros((L,), jnp.int32) + rank
            hv = _gather(hw[...], rank_v - hoff)
            rank_f = rank_v.astype(jnp.float32)
            wv = jnp.where(rank_v > 0, hv / jnp.maximum(rank_f, 1.0), 0.0)
            contrib = contrib + jnp.where(iota == r, wv * (hinge * (1.0 / B)), 0.0)

        ov[...] = contrib
        pltpu.sync_copy(ov, out_hbm.at[wid])

    return wrpl


_WRPL = _make_kernel()


def kernel(score, target):
    parts = _WRPL(score.reshape(-1), target, jnp.asarray(_HARM))
    return jnp.sum(parts)


# speedup vs baseline: 49.7332x; 1.2552x over previous
import functools

import numpy as np
import jax
import jax.numpy as jnp
from jax import lax
from jax.experimental import pallas as pl
from jax.experimental.pallas import tpu as pltpu
from jax.experimental.pallas import tpu_sc as plsc

B = 128
N = 100000
L = 16
CHUNK = 20000
NCHUNK = N // CHUNK
NVREG = CHUNK // L
UNROLL = 5

_info = plsc.get_sparse_core_info()
NC, NS = _info.num_cores, _info.num_subcores
NW = NC * NS
RPW = B // NW
TOT = RPW * NCHUNK

_HARM = np.concatenate([
    np.zeros(1, np.float32),
    np.cumsum((1.0 / np.arange(1, N, dtype=np.float32)), dtype=np.float32),
]).astype(np.float32)


def _gather(vec, idx):
    return vec.at[idx].get(mode="promise_in_bounds")


def _make_kernel():
    mesh = plsc.VectorSubcoreMesh(core_axis_name="c", subcore_axis_name="s")

    @functools.partial(
        pl.kernel,
        mesh=mesh,
        compiler_params=pltpu.CompilerParams(needs_layout_passes=False),
        out_type=jax.ShapeDtypeStruct((NW, L), jnp.float32),
        scratch_types=[
            pltpu.VMEM((CHUNK,), jnp.float32),
            pltpu.VMEM((CHUNK,), jnp.float32),
            pltpu.VMEM((B,), jnp.int32),
            pltpu.VMEM((RPW, L), jnp.float32),
            pltpu.VMEM((L,), jnp.float32),
            pltpu.VMEM((L,), jnp.float32),
            pltpu.SemaphoreType.DMA,
            pltpu.SemaphoreType.DMA,
        ],
    )
    def wrpl(score_hbm, target_hbm, harm_hbm, out_hbm,
             buf0, buf1, tgt_v, gtw, hw, ov, sem0, sem1):
        wid = lax.axis_index("s") * NC + lax.axis_index("c")
        row0 = wid * RPW
        iota = lax.iota(jnp.int32, L)

        pltpu.sync_copy(target_hbm, tgt_v)

        bufs = (buf0, buf1)
        sems = (sem0, sem1)

        def issue(k, slot):
            b = row0 + (k // NCHUNK)
            off = pl.multiple_of(b * N + (k % NCHUNK) * CHUNK, 8)
            return pltpu.async_copy(score_hbm.at[pl.ds(off, CHUNK)],
                                    bufs[slot], sems[slot])

        handles = [issue(0, 0), None]

        tvs, bss, gtvs, ctbs = [], [], [], []
        for r in range(RPW):
            b = row0 + r
            wb = (b // L) * L
            twin = tgt_v[pl.ds(wb, L)]
            tv = _gather(twin, jnp.zeros((L,), jnp.int32) + (b - wb))
            t_s = jnp.max(tv)
            bs = (t_s // L) * L
            pltpu.sync_copy(
                score_hbm.at[pl.ds(pl.multiple_of(b * N + bs, 8), L)],
                gtw.at[r])
            win = gtw[r]
            gtv = _gather(win, tv - bs)
            ctb = jnp.sum(((win == gtv) & (iota < (tv - bs))).astype(jnp.int32))
            tvs.append(tv)
            bss.append(bs)
            gtvs.append(gtv)
            ctbs.append(ctb)

        contrib = jnp.zeros((L,), jnp.float32)
        for r in range(RPW):
            bs, gtv = bss[r], gtvs[r]
            gtm1 = gtv - 1.0
            zero_i = jnp.zeros((L,), jnp.int32)
            zero_f = jnp.zeros((L,), jnp.float32)
            accs = tuple((zero_i, zero_f) for _ in range(UNROLL))
            for c in range(NCHUNK):
                k = r * NCHUNK + c
                slot = k % 2
                handles[slot].wait()
                if k + 1 < TOT:
                    handles[1 - slot] = issue(k + 1, 1 - slot)
                buf = bufs[slot]
                base = c * CHUNK

                def body(i, carry, buf=buf, base=base, bs=bs, gtv=gtv, gtm1=gtm1):
                    out = []
                    for u in range(UNROLL):
                        cg, hh = carry[u]
                        g = base + (i + u) * L
                        v = buf[pl.ds((i + u) * L, L)]
                        mgt = v > gtv
                        mge = v >= gtv
                        m = jnp.where(g < bs, mge, mgt)
                        cg = cg + m.astype(jnp.int32)
                        hh = hh + jnp.maximum(v - gtm1, 0.0)
                        out.append((cg, hh))
                    return tuple(out)

                accs = plsc.parallel_loop(
                    0, NVREG, UNROLL, unroll=1, carry=accs)(body)

            rank = ctbs[r]
            hh_t = zero_f
            for u in range(UNROLL):
                rank = rank + jnp.sum(accs[u][0])
                hh_t = hh_t + accs[u][1]
            hinge = jnp.sum(hh_t) - 1.0

            hoff = jnp.minimum((rank // 8) * 8, N - L)
            pltpu.sync_copy(harm_hbm.at[pl.ds(pl.multiple_of(hoff, 8), L)], hw)
            rank_v = jnp.zeros((L,), jnp.int32) + rank
            hv = _gather(hw[...], rank_v - hoff)
            rank_f = rank_v.astype(jnp.float32)
            wv = jnp.where(rank_v > 0, hv / jnp.maximum(rank_f, 1.0), 0.0)
            contrib = contrib + jnp.where(iota == r, wv * (hinge * (1.0 / B)), 0.0)

        ov[...] = contrib
        pltpu.sync_copy(ov, out_hbm.at[wid])

    return wrpl


_WRPL = _make_kernel()


def kernel(score, target):
    parts = _WRPL(score.reshape(-1), target, jnp.asarray(_HARM))
    return jnp.sum(parts)


# speedup vs baseline: 49.9264x; 1.0039x over previous
import functools

import numpy as np
import jax
import jax.numpy as jnp
from jax import lax
from jax.experimental import pallas as pl
from jax.experimental.pallas import tpu as pltpu
from jax.experimental.pallas import tpu_sc as plsc

B = 128
N = 100000
L = 16
CHUNK = 20000
NCHUNK = N // CHUNK
NVREG = CHUNK // L
UNROLL = 10

_info = plsc.get_sparse_core_info()
NC, NS = _info.num_cores, _info.num_subcores
NW = NC * NS
RPW = B // NW
TOT = RPW * NCHUNK

_HARM = np.concatenate([
    np.zeros(1, np.float32),
    np.cumsum((1.0 / np.arange(1, N, dtype=np.float32)), dtype=np.float32),
]).astype(np.float32)


def _gather(vec, idx):
    return vec.at[idx].get(mode="promise_in_bounds")


def _make_kernel():
    mesh = plsc.VectorSubcoreMesh(core_axis_name="c", subcore_axis_name="s")

    @functools.partial(
        pl.kernel,
        mesh=mesh,
        compiler_params=pltpu.CompilerParams(needs_layout_passes=False),
        out_type=jax.ShapeDtypeStruct((NW, L), jnp.float32),
        scratch_types=[
            pltpu.VMEM((CHUNK,), jnp.float32),
            pltpu.VMEM((CHUNK,), jnp.float32),
            pltpu.VMEM((B,), jnp.int32),
            pltpu.VMEM((RPW, L), jnp.float32),
            pltpu.VMEM((L,), jnp.float32),
            pltpu.VMEM((L,), jnp.float32),
            pltpu.SemaphoreType.DMA,
            pltpu.SemaphoreType.DMA,
        ],
    )
    def wrpl(score_hbm, target_hbm, harm_hbm, out_hbm,
             buf0, buf1, tgt_v, gtw, hw, ov, sem0, sem1):
        wid = lax.axis_index("s") * NC + lax.axis_index("c")
        row0 = wid * RPW
        iota = lax.iota(jnp.int32, L)

        pltpu.sync_copy(target_hbm, tgt_v)

        bufs = (buf0, buf1)
        sems = (sem0, sem1)

        def issue(k, slot):
            b = row0 + (k // NCHUNK)
            off = pl.multiple_of(b * N + (k % NCHUNK) * CHUNK, 8)
            return pltpu.async_copy(score_hbm.at[pl.ds(off, CHUNK)],
                                    bufs[slot], sems[slot])

        handles = [issue(0, 0), None]

        tvs, bss, gtvs, ctbs = [], [], [], []
        for r in range(RPW):
            b = row0 + r
            wb = (b // L) * L
            twin = tgt_v[pl.ds(wb, L)]
            tv = _gather(twin, jnp.zeros((L,), jnp.int32) + (b - wb))
            t_s = jnp.max(tv)
            bs = (t_s // L) * L
            pltpu.sync_copy(
                score_hbm.at[pl.ds(pl.multiple_of(b * N + bs, 8), L)],
                gtw.at[r])
            win = gtw[r]
            gtv = _gather(win, tv - bs)
            ctb = jnp.sum(((win == gtv) & (iota < (tv - bs))).astype(jnp.int32))
            tvs.append(tv)
            bss.append(bs)
            gtvs.append(gtv)
            ctbs.append(ctb)

        contrib = jnp.zeros((L,), jnp.float32)
        for r in range(RPW):
            bs, gtv = bss[r], gtvs[r]
            gtm1 = gtv - 1.0
            zero_i = jnp.zeros((L,), jnp.int32)
            zero_f = jnp.zeros((L,), jnp.float32)
            accs = tuple((zero_i, zero_f) for _ in range(UNROLL))
            for c in range(NCHUNK):
                k = r * NCHUNK + c
                slot = k % 2
                handles[slot].wait()
                if k + 1 < TOT:
                    handles[1 - slot] = issue(k + 1, 1 - slot)
                buf = bufs[slot]
                base = c * CHUNK

                def body(i, carry, buf=buf, base=base, bs=bs, gtv=gtv, gtm1=gtm1):
                    out = []
                    for u in range(UNROLL):
                        cg, hh = carry[u]
                        g = base + (i + u) * L
                        v = buf[pl.ds((i + u) * L, L)]
                        mgt = v > gtv
                        mge = v >= gtv
                        m = jnp.where(g < bs, mge, mgt)
                        cg = cg + plsc.all_reduce_population_count(m)
                        hh = hh + jnp.maximum(v - gtm1, 0.0)
                        out.append((cg, hh))
                    return tuple(out)

                accs = plsc.parallel_loop(
                    0, NVREG, UNROLL, unroll=1, carry=accs)(body)

            rank = ctbs[r]
            hh_t = zero_f
            for u in range(UNROLL):
                rank = rank + jnp.max(accs[u][0])
                hh_t = hh_t + accs[u][1]
            hinge = jnp.sum(hh_t) - 1.0

            hoff = jnp.minimum((rank // 8) * 8, N - L)
            pltpu.sync_copy(harm_hbm.at[pl.ds(pl.multiple_of(hoff, 8), L)], hw)
            rank_v = jnp.zeros((L,), jnp.int32) + rank
            hv = _gather(hw[...], rank_v - hoff)
            rank_f = rank_v.astype(jnp.float32)
            wv = jnp.where(rank_v > 0, hv / jnp.maximum(rank_f, 1.0), 0.0)
            contrib = contrib + jnp.where(iota == r, wv * (hinge * (1.0 / B)), 0.0)

        ov[...] = contrib
        pltpu.sync_copy(ov, out_hbm.at[wid])

    return wrpl


_WRPL = _make_kernel()


def kernel(score, target):
    parts = _WRPL(score.reshape(-1), target, jnp.asarray(_HARM))
    return jnp.sum(parts)
